# baseline (speedup 1.0000x reference)
import functools
import jax
import jax.numpy as jnp
from jax import lax
from jax.experimental import pallas as pl
from jax.experimental.pallas import tpu as pltpu
from jax.experimental.pallas import tpu_sc as plsc

_CH = 8


def kernel(x):
    B, C, D = x.shape
    info = plsc.get_sparse_core_info()
    NC, NS = info.num_cores, info.num_subcores
    NW = NC * NS
    rows_per_w = B // NW
    n_chunks = rows_per_w // _CH
    nd = D // 16

    mesh = plsc.VectorSubcoreMesh(core_axis_name="c", subcore_axis_name="s")

    _gdn = lax.GatherDimensionNumbers(
        offset_dims=(), collapsed_slice_dims=(0,), start_index_map=(0,))

    def _perm(v, idx):
        return lax.gather(v, idx[:, None], _gdn, slice_sizes=(1,),
                          mode=lax.GatherScatterMode.PROMISE_IN_BOUNDS)

    def _bfly(v, op, iot):
        for d in (1, 2, 4, 8):
            v = op(v, _perm(v, iot ^ d))
        return v

    @functools.partial(
        pl.kernel,
        mesh=mesh,
        out_type=[
            jax.ShapeDtypeStruct((B, D), jnp.float32),
            jax.ShapeDtypeStruct((B, C, D), jnp.float32),
        ],
        scratch_types=[
            pltpu.VMEM((_CH, C, D), jnp.float32),
            pltpu.VMEM((_CH, C, D), jnp.float32),
            pltpu.VMEM((_CH, D), jnp.float32),
        ],
        compiler_params=pltpu.CompilerParams(use_tc_tiling_on_sc=True),
    )
    def run(x_hbm, logits_hbm, latent_hbm, x_buf, out_buf, log_buf):
        wid = lax.axis_index("s") * NC + lax.axis_index("c")
        w_base = wid * rows_per_w
        iot = lax.broadcasted_iota(jnp.int32, (16,), 0)

        def chunk_body(ch, carry):
            base = w_base + ch * _CH
            pltpu.sync_copy(x_hbm.at[pl.ds(base, _CH)], x_buf)

            def row_body(r, carry2):
                ss = []
                for k in range(nd):
                    acc = x_buf[r, 0, pl.ds(k * 16, 16)]
                    acc = acc * acc
                    for c in range(1, C):
                        v = x_buf[r, c, pl.ds(k * 16, 16)]
                        acc = acc + v * v
                    ss.append(acc)
                for k in range(nd):
                    s = ss[k]
                    i = lax.bitcast_convert_type(s, jnp.int32)
                    y = lax.bitcast_convert_type(
                        jnp.int32(0x5F3759DF) - (i >> 1), jnp.float32)
                    for _ in range(3):
                        y = y * (1.5 - 0.5 * s * y * y)
                    log_buf[r, pl.ds(k * 16, 16)] = jnp.where(
                        s > 0.0, s * y, 0.0)
                m = ss[0]
                for k in range(1, nd):
                    m = jnp.maximum(m, ss[k])
                gm = _bfly(m, jnp.maximum, iot)
                cand = jnp.where(ss[0] == gm, iot, D)
                for k in range(1, nd):
                    cand = jnp.minimum(
                        cand, jnp.where(ss[k] == gm, iot + 16 * k, D))
                gi = _bfly(cand, jnp.minimum, iot)
                masks = [
                    jnp.where((iot + 16 * k) == gi, 1.0, 0.0).astype(jnp.float32)
                    for k in range(nd)
                ]
                for c in range(C):
                    for k in range(nd):
                        out_buf[r, c, pl.ds(k * 16, 16)] = (
                            x_buf[r, c, pl.ds(k * 16, 16)] * masks[k])
                return carry2

            lax.fori_loop(0, _CH, row_body, 0)
            pltpu.sync_copy(out_buf, latent_hbm.at[pl.ds(base, _CH)])
            pltpu.sync_copy(log_buf, logits_hbm.at[pl.ds(base, _CH)])
            return carry

        lax.fori_loop(0, n_chunks, chunk_body, 0)

    logits, latent = run(x)
    return (logits, latent.reshape(B, C * D))


# speedup vs baseline: 1.1445x; 1.1445x over previous
import functools
import jax
import jax.numpy as jnp
from jax import lax
from jax.experimental import pallas as pl
from jax.experimental.pallas import tpu as pltpu
from jax.experimental.pallas import tpu_sc as plsc

_NB = 128
_NG = _NB // 16
_DS = 8
_CS = 4


def kernel(x):
    B, C, D = x.shape
    F = C * D
    xt = jnp.transpose(x, (1, 2, 0))
    info = plsc.get_sparse_core_info()
    NC, NS = info.num_cores, info.num_subcores
    NW = NC * NS
    b_per_w = B // NW
    mesh = plsc.VectorSubcoreMesh(core_axis_name="c", subcore_axis_name="s")

    @functools.partial(
        pl.kernel,
        mesh=mesh,
        out_type=[
            jax.ShapeDtypeStruct((D, B), jnp.float32),
            jax.ShapeDtypeStruct((F, B), jnp.float32),
        ],
        scratch_types=[
            pltpu.VMEM((_CS, D, _NB), jnp.float32),
            pltpu.VMEM((_CS * D, _NB), jnp.float32),
            pltpu.VMEM((D, _NB), jnp.float32),
            pltpu.VMEM((D, _NB), jnp.float32),
            pltpu.VMEM((_NG, 16), jnp.int32),
        ],
        compiler_params=pltpu.CompilerParams(use_tc_tiling_on_sc=True),
    )
    def run(xt_hbm, logt_hbm, latt_hbm, x_buf, out_buf, s_buf, logt_buf,
            gi_buf):
        wid = lax.axis_index("s") * NC + lax.axis_index("c")
        base = wid * b_per_w

        def chunk(i, carry):
            b0 = base + i * _NB

            def a_slab(cs, carry2):
                pltpu.sync_copy(
                    xt_hbm.at[pl.ds(cs * _CS, _CS), :, pl.ds(b0, _NB)],
                    x_buf)

                def a_body(j, carry3):
                    d = j // _NG
                    g = j % _NG
                    sl = pl.ds(g * 16, 16)
                    acc = x_buf[0, d, sl]
                    acc = acc * acc
                    for c in range(1, _CS):
                        v = x_buf[c, d, sl]
                        acc = acc + v * v

                    @pl.when(cs > 0)
                    def _():
                        s_buf[d, sl] = s_buf[d, sl] + acc

                    @pl.when(cs == 0)
                    def _():
                        s_buf[d, sl] = acc

                    return carry3

                lax.fori_loop(0, D * _NG, a_body, 0)
                return carry2

            lax.fori_loop(0, C // _CS, a_slab, 0)

            def fin_body(j, carry2):
                d = j // _NG
                g = j % _NG
                sl = pl.ds(g * 16, 16)
                s = s_buf[d, sl]
                iv = lax.bitcast_convert_type(s, jnp.int32)
                y = lax.bitcast_convert_type(
                    jnp.int32(0x5F3759DF) - (iv >> 1), jnp.float32)
                for _ in range(3):
                    y = y * (1.5 - 0.5 * s * y * y)
                logt_buf[d, sl] = jnp.where(s > 0.0, s * y, 0.0)
                return carry2

            lax.fori_loop(0, D * _NG, fin_body, 0)

            for g in range(_NG):
                sl = pl.ds(g * 16, 16)
                ss = [s_buf[d, sl] for d in range(D)]
                m = ss[0]
                for d in range(1, D):
                    m = jnp.maximum(m, ss[d])
                cand = jnp.full((16,), D, jnp.int32)
                for d in range(D - 1, -1, -1):
                    cand = jnp.where(ss[d] == m, d, cand)
                gi_buf[g, :] = cand

            pltpu.sync_copy(logt_buf, logt_hbm.at[:, pl.ds(b0, _NB)])

            def b_slab(cs, carry2):
                pltpu.sync_copy(
                    xt_hbm.at[pl.ds(cs * _CS, _CS), :, pl.ds(b0, _NB)],
                    x_buf)

                def b_body(j, carry3):
                    d = j // _NG
                    g = j % _NG
                    sl = pl.ds(g * 16, 16)
                    keep = gi_buf[g, :] == d
                    for c in range(_CS):
                        v = x_buf[c, d, sl]
                        out_buf[c * D + d, sl] = jnp.where(keep, v, 0.0)
                    return carry3

                lax.fori_loop(0, D * _NG, b_body, 0)
                pltpu.sync_copy(
                    out_buf,
                    latt_hbm.at[pl.ds(cs * _CS * D, _CS * D),
                                pl.ds(b0, _NB)])
                return carry2

            lax.fori_loop(0, C // _CS, b_slab, 0)
            return carry

        lax.fori_loop(0, b_per_w // _NB, chunk, 0)

    logt, latt = run(xt)
    return (jnp.transpose(logt), jnp.transpose(latt).reshape(B, F))


# speedup vs baseline: 1.6006x; 1.3985x over previous
import functools
import jax
import jax.numpy as jnp
from jax import lax
from jax.experimental import pallas as pl
from jax.experimental.pallas import tpu as pltpu
from jax.experimental.pallas import tpu_sc as plsc

_NB = 128
_NG = _NB // 16
_DS = 8
_CS = 4


def kernel(x):
    B, C, D = x.shape
    F = C * D
    xt = jnp.transpose(x, (1, 2, 0))
    info = plsc.get_sparse_core_info()
    NC, NS = info.num_cores, info.num_subcores
    NW = NC * NS
    b_per_w = B // NW
    mesh = plsc.VectorSubcoreMesh(core_axis_name="c", subcore_axis_name="s")

    @functools.partial(
        pl.kernel,
        mesh=mesh,
        out_type=[
            jax.ShapeDtypeStruct((D, B), jnp.float32),
            jax.ShapeDtypeStruct((F, B), jnp.float32),
        ],
        scratch_types=[
            pltpu.VMEM((C, _DS, _NB), jnp.float32),
            pltpu.VMEM((_CS, D, _NB), jnp.float32),
            pltpu.VMEM((_CS * D, _NB), jnp.float32),
            pltpu.VMEM((D, _NB), jnp.float32),
            pltpu.VMEM((D, _NB), jnp.float32),
            pltpu.VMEM((_NG, 16), jnp.int32),
        ],
        compiler_params=pltpu.CompilerParams(use_tc_tiling_on_sc=True),
    )
    def run(xt_hbm, logt_hbm, latt_hbm, xa_buf, xb_buf, out_buf, s_buf,
            logt_buf, gi_buf):
        wid = lax.axis_index("s") * NC + lax.axis_index("c")
        base = wid * b_per_w

        def chunk(i, carry):
            b0 = base + i * _NB

            def a_slab(sd, carry2):
                pltpu.sync_copy(
                    xt_hbm.at[:, pl.ds(sd * _DS, _DS), pl.ds(b0, _NB)],
                    xa_buf)

                def a_body(dl, carry3):
                    d = sd * _DS + dl
                    for g in range(_NG):
                        sl = pl.ds(g * 16, 16)
                        acc = xa_buf[0, dl, sl]
                        acc = acc * acc
                        for c in range(1, C):
                            v = xa_buf[c, dl, sl]
                            acc = acc + v * v
                        s_buf[d, sl] = acc
                        iv = lax.bitcast_convert_type(acc, jnp.int32)
                        y = lax.bitcast_convert_type(
                            jnp.int32(0x5F3759DF) - (iv >> 1), jnp.float32)
                        for _ in range(3):
                            y = y * (1.5 - 0.5 * acc * y * y)
                        logt_buf[d, sl] = jnp.where(acc > 0.0, acc * y, 0.0)
                    return carry3

                lax.fori_loop(0, _DS, a_body, 0)
                return carry2

            lax.fori_loop(0, D // _DS, a_slab, 0)

            for g in range(_NG):
                sl = pl.ds(g * 16, 16)
                ss = [s_buf[d, sl] for d in range(D)]
                m = ss[0]
                for d in range(1, D):
                    m = jnp.maximum(m, ss[d])
                cand = jnp.full((16,), D, jnp.int32)
                for d in range(D - 1, -1, -1):
                    cand = jnp.where(ss[d] == m, d, cand)
                gi_buf[g, :] = cand

            pltpu.sync_copy(logt_buf, logt_hbm.at[:, pl.ds(b0, _NB)])

            def b_slab(cs, carry2):
                pltpu.sync_copy(
                    xt_hbm.at[pl.ds(cs * _CS, _CS), :, pl.ds(b0, _NB)],
                    xb_buf)

                def b_body(d, carry3):
                    for g in range(_NG):
                        sl = pl.ds(g * 16, 16)
                        keep = gi_buf[g, :] == d
                        for c in range(_CS):
                            v = xb_buf[c, d, sl]
                            out_buf[c * D + d, sl] = jnp.where(keep, v, 0.0)
                    return carry3

                lax.fori_loop(0, D, b_body, 0)
                pltpu.sync_copy(
                    out_buf,
                    latt_hbm.at[pl.ds(cs * _CS * D, _CS * D),
                                pl.ds(b0, _NB)])
                return carry2

            lax.fori_loop(0, C // _CS, b_slab, 0)
            return carry

        lax.fori_loop(0, b_per_w // _NB, chunk, 0)

    logt, latt = run(xt)
    return (jnp.transpose(logt), jnp.transpose(latt).reshape(B, F))


# speedup vs baseline: 3.6375x; 2.2726x over previous
import functools
import jax
import jax.numpy as jnp
from jax import lax
from jax.experimental import pallas as pl
from jax.experimental.pallas import tpu as pltpu
from jax.experimental.pallas import tpu_sc as plsc

_NB = 128
_NG = _NB // 16
_CH = 16
_DS = 8
_CS = 2


def kernel(x):
    B, C, D = x.shape
    F = C * D
    xt = jnp.transpose(x, (1, 2, 0))
    info = plsc.get_sparse_core_info()
    NC, NS = info.num_cores, info.num_subcores
    NW = NC * NS
    b_per_w = B // NW
    nd = D // _DS
    nk = C // _CS
    mesh = plsc.VectorSubcoreMesh(core_axis_name="c", subcore_axis_name="s")

    @functools.partial(
        pl.kernel,
        mesh=mesh,
        out_type=[
            jax.ShapeDtypeStruct((D, B), jnp.float32),
            jax.ShapeDtypeStruct((B, F), jnp.float32),
        ],
        scratch_types=[
            pltpu.VMEM((_CH, _DS, _NB), jnp.float32),
            pltpu.VMEM((_CH, _DS, _NB), jnp.float32),
            pltpu.VMEM((_CS, D, _NB), jnp.float32),
            pltpu.VMEM((_CS, D, _NB), jnp.float32),
            pltpu.VMEM((_NB, _CS * D), jnp.float32),
            pltpu.VMEM((_NB, _CS * D), jnp.float32),
            pltpu.VMEM((D, _NB), jnp.float32),
            pltpu.VMEM((_NG, 16), jnp.int32),
            pltpu.SemaphoreType.DMA,
            pltpu.SemaphoreType.DMA,
            pltpu.SemaphoreType.DMA,
            pltpu.SemaphoreType.DMA,
            pltpu.SemaphoreType.DMA,
            pltpu.SemaphoreType.DMA,
        ],
        compiler_params=pltpu.CompilerParams(
            use_tc_tiling_on_sc=True, needs_layout_passes=False),
    )
    def run(xt_hbm, logt_hbm, lat_hbm, xa0, xa1, xb0, xb1, nb0, nb1,
            s_buf, gi_buf, sa0, sa1, sb0, sb1, so0, so1):
        wid = lax.axis_index("s") * NC + lax.axis_index("c")
        base = wid * b_per_w
        lanes = lax.iota(jnp.int32, 16)
        zero16 = jnp.zeros((16,), jnp.float32)

        def a_src(b0, j, ch):
            return xt_hbm.at[pl.ds(ch * _CH, _CH),
                             pl.ds(j * _DS, _DS), pl.ds(b0, _NB)]

        def b_src(b0, k):
            return xt_hbm.at[pl.ds(k * _CS, _CS), :, pl.ds(b0, _NB)]

        def lat_dst(b0, k):
            return lat_hbm.at[pl.ds(b0, _NB), pl.ds(k * (_CS * D), _CS * D)]

        def znb(r, carry):
            for j in range(_CS * D // 16):
                nb0[r, pl.ds(j * 16, 16)] = zero16
                nb1[r, pl.ds(j * 16, 16)] = zero16
            return carry

        lax.fori_loop(0, _NB, znb, 0)

        def fill_nb(nb, xb):
            for cl in range(_CS):
                clv = jnp.full((16,), cl, jnp.int32)
                for g in range(_NG):
                    giv = gi_buf[g, :]
                    rows = lanes + g * 16
                    vals = plsc.load_gather(xb, [clv, giv, rows])
                    plsc.store_scatter(nb, [rows, giv + cl * D], vals)

        def chunk(i, carry):
            b0 = base + i * _NB

            pltpu.async_copy(a_src(b0, 0, 0), xa0, sa0)

            def half(buf, j, ch):
                def dl_body(dl, c3):
                    d = j * _DS + dl
                    for g in range(_NG):
                        sl = pl.ds(g * 16, 16)
                        v = buf[0, dl, sl]
                        acc = v * v
                        for c in range(1, _CH):
                            v = buf[c, dl, sl]
                            acc = acc + v * v
                        if ch == 0:
                            s_buf[d, sl] = acc
                        else:
                            s_buf[d, sl] = s_buf[d, sl] + acc
                    return c3

                lax.fori_loop(0, _DS, dl_body, 0)

            def aj(j, carry2):
                pltpu.async_copy(a_src(b0, j, 1), xa1, sa1)
                pltpu.make_async_copy(a_src(b0, j, 0), xa0, sa0).wait()
                half(xa0, j, 0)

                @pl.when(j < nd - 1)
                def _():
                    pltpu.async_copy(a_src(b0, j + 1, 0), xa0, sa0)

                pltpu.make_async_copy(a_src(b0, j, 1), xa1, sa1).wait()
                half(xa1, j, 1)
                return carry2

            lax.fori_loop(0, nd, aj, 0)

            pltpu.async_copy(b_src(b0, 0), xb0, sb0)

            for g in range(_NG):
                sl = pl.ds(g * 16, 16)
                ss = [s_buf[d, sl] for d in range(D)]
                m = ss[0]
                for d in range(1, D):
                    m = jnp.maximum(m, ss[d])
                cand = jnp.full((16,), D, jnp.int32)
                for d in range(D - 1, -1, -1):
                    cand = jnp.where(ss[d] == m, d, cand)
                gi_buf[g, :] = cand

            def nl(d, c3):
                for g in range(_NG):
                    sl = pl.ds(g * 16, 16)
                    acc = s_buf[d, sl]
                    iv = lax.bitcast_convert_type(acc, jnp.int32)
                    y = lax.bitcast_convert_type(
                        jnp.int32(0x5F3759DF) - (iv >> 1), jnp.float32)
                    for _ in range(3):
                        y = y * (1.5 - 0.5 * acc * y * y)
                    s_buf[d, sl] = jnp.where(acc > 0.0, acc * y, 0.0)
                return c3

            lax.fori_loop(0, D, nl, 0)
            pltpu.sync_copy(s_buf, logt_hbm.at[:, pl.ds(b0, _NB)])

            def bj(jj, carry2):
                k0 = 2 * jj
                pltpu.async_copy(b_src(b0, k0 + 1), xb1, sb1)
                pltpu.make_async_copy(b_src(b0, k0), xb0, sb0).wait()

                @pl.when(jj > 0)
                def _():
                    pltpu.make_async_copy(nb0, lat_dst(b0, k0 - 2),
                                          so0).wait()

                fill_nb(nb0, xb0)
                pltpu.async_copy(nb0, lat_dst(b0, k0), so0)

                @pl.when(jj < nk // 2 - 1)
                def _():
                    pltpu.async_copy(b_src(b0, k0 + 2), xb0, sb0)

                pltpu.make_async_copy(b_src(b0, k0 + 1), xb1, sb1).wait()

                @pl.when(jj > 0)
                def _():
                    pltpu.make_async_copy(nb1, lat_dst(b0, k0 - 1),
                                          so1).wait()

                fill_nb(nb1, xb1)
                pltpu.async_copy(nb1, lat_dst(b0, k0 + 1), so1)
                return carry2

            lax.fori_loop(0, nk // 2, bj, 0)

            pltpu.make_async_copy(nb0, lat_dst(b0, nk - 2), so0).wait()
            pltpu.make_async_copy(nb1, lat_dst(b0, nk - 1), so1).wait()
            for g in range(_NG):
                giv = gi_buf[g, :]
                rows = lanes + g * 16
                for cl in range(_CS):
                    plsc.store_scatter(nb0, [rows, giv + cl * D], zero16)
                    plsc.store_scatter(nb1, [rows, giv + cl * D], zero16)
            return carry

        lax.fori_loop(0, b_per_w // _NB, chunk, 0)

    logt, lat = run(xt)
    return (jnp.transpose(logt), lat)


# speedup vs baseline: 3.9798x; 1.0941x over previous
import functools
import jax
import jax.numpy as jnp
from jax import lax
from jax.experimental import pallas as pl
from jax.experimental.pallas import tpu as pltpu
from jax.experimental.pallas import tpu_sc as plsc

_NB = 128
_NG = _NB // 16
_CH = 16
_DS = 8
_CS = 2


def kernel(x):
    B, C, D = x.shape
    F = C * D
    xt = jnp.transpose(x, (1, 2, 0))
    info = plsc.get_sparse_core_info()
    NC, NS = info.num_cores, info.num_subcores
    NW = NC * NS
    b_per_w = B // NW
    nd = D // _DS
    nk = C // _CS
    mesh = plsc.VectorSubcoreMesh(core_axis_name="c", subcore_axis_name="s")

    @functools.partial(
        pl.kernel,
        mesh=mesh,
        out_type=[
            jax.ShapeDtypeStruct((D, B), jnp.float32),
            jax.ShapeDtypeStruct((B, F), jnp.float32),
        ],
        scratch_types=[
            pltpu.VMEM((_CH, _DS, _NB), jnp.float32),
            pltpu.VMEM((_CH, _DS, _NB), jnp.float32),
            pltpu.VMEM((_CS, D, _NB), jnp.float32),
            pltpu.VMEM((_CS, D, _NB), jnp.float32),
            pltpu.VMEM((_NB, _CS * D), jnp.float32),
            pltpu.VMEM((_NB, _CS * D), jnp.float32),
            pltpu.VMEM((D, _NB), jnp.float32),
            pltpu.VMEM((_NG, 16), jnp.int32),
            pltpu.SemaphoreType.DMA,
            pltpu.SemaphoreType.DMA,
            pltpu.SemaphoreType.DMA,
            pltpu.SemaphoreType.DMA,
            pltpu.SemaphoreType.DMA,
            pltpu.SemaphoreType.DMA,
        ],
        compiler_params=pltpu.CompilerParams(
            use_tc_tiling_on_sc=True, needs_layout_passes=False),
    )
    def run(xt_hbm, logt_hbm, lat_hbm, xa0, xa1, xb0, xb1, nb0, nb1,
            s_buf, gi_buf, sa0, sa1, sb0, sb1, so0, so1):
        wid = lax.axis_index("s") * NC + lax.axis_index("c")
        base = wid * b_per_w
        lanes = lax.iota(jnp.int32, 16)
        zero16 = jnp.zeros((16,), jnp.float32)

        def a_src(b0, j, ch):
            return xt_hbm.at[pl.ds(ch * _CH, _CH),
                             pl.ds(j * _DS, _DS), pl.ds(b0, _NB)]

        def b_src(b0, k):
            return xt_hbm.at[pl.ds(k * _CS, _CS), :, pl.ds(b0, _NB)]

        def lat_dst(b0, k):
            return lat_hbm.at[pl.ds(b0, _NB), pl.ds(k * (_CS * D), _CS * D)]

        def znb(r, carry):
            for j in range(_CS * D // 16):
                nb0[r, pl.ds(j * 16, 16)] = zero16
                nb1[r, pl.ds(j * 16, 16)] = zero16
            return carry

        lax.fori_loop(0, _NB, znb, 0)

        def fill_nb(nb, xb):
            for cl in range(_CS):
                clv = jnp.full((16,), cl, jnp.int32)
                for g in range(_NG):
                    giv = gi_buf[g, :]
                    rows = lanes + g * 16
                    vals = plsc.load_gather(xb, [clv, giv, rows])
                    plsc.store_scatter(nb, [rows, giv + cl * D], vals)

        def chunk(i, carry):
            b0 = base + i * _NB

            pltpu.async_copy(a_src(b0, 0, 0), xa0, sa0)

            def half(buf, j, ch):
                def dl_body(dl2, c3):
                    for du in range(2):
                        dl = dl2 * 2 + du
                        d = j * _DS + dl
                        for g in range(_NG):
                            sl = pl.ds(g * 16, 16)
                            v0 = buf[0, dl, sl]
                            v1 = buf[1, dl, sl]
                            v2 = buf[2, dl, sl]
                            v3 = buf[3, dl, sl]
                            a0 = v0 * v0
                            a1 = v1 * v1
                            a2 = v2 * v2
                            a3 = v3 * v3
                            for c in range(4, _CH, 4):
                                v0 = buf[c, dl, sl]
                                v1 = buf[c + 1, dl, sl]
                                v2 = buf[c + 2, dl, sl]
                                v3 = buf[c + 3, dl, sl]
                                a0 = a0 + v0 * v0
                                a1 = a1 + v1 * v1
                                a2 = a2 + v2 * v2
                                a3 = a3 + v3 * v3
                            acc = (a0 + a1) + (a2 + a3)
                            if ch == 0:
                                s_buf[d, sl] = acc
                            else:
                                s_buf[d, sl] = s_buf[d, sl] + acc
                    return c3

                lax.fori_loop(0, _DS // 2, dl_body, 0)

            def aj(j, carry2):
                pltpu.async_copy(a_src(b0, j, 1), xa1, sa1)
                pltpu.make_async_copy(a_src(b0, j, 0), xa0, sa0).wait()
                half(xa0, j, 0)

                @pl.when(j < nd - 1)
                def _():
                    pltpu.async_copy(a_src(b0, j + 1, 0), xa0, sa0)

                pltpu.make_async_copy(a_src(b0, j, 1), xa1, sa1).wait()
                half(xa1, j, 1)
                return carry2

            lax.fori_loop(0, nd, aj, 0)

            pltpu.async_copy(b_src(b0, 0), xb0, sb0)

            for g in range(_NG):
                sl = pl.ds(g * 16, 16)
                ss = [s_buf[d, sl] for d in range(D)]
                m = ss[0]
                for d in range(1, D):
                    m = jnp.maximum(m, ss[d])
                cand = jnp.full((16,), D, jnp.int32)
                for d in range(D - 1, -1, -1):
                    cand = jnp.where(ss[d] == m, d, cand)
                gi_buf[g, :] = cand

            def nl(d, c3):
                for g in range(_NG):
                    sl = pl.ds(g * 16, 16)
                    acc = s_buf[d, sl]
                    iv = lax.bitcast_convert_type(acc, jnp.int32)
                    y = lax.bitcast_convert_type(
                        jnp.int32(0x5F3759DF) - (iv >> 1), jnp.float32)
                    for _ in range(2):
                        y = y * (1.5 - 0.5 * acc * y * y)
                    s_buf[d, sl] = jnp.where(acc > 0.0, acc * y, 0.0)
                return c3

            lax.fori_loop(0, D, nl, 0)
            pltpu.sync_copy(s_buf, logt_hbm.at[:, pl.ds(b0, _NB)])

            def bj(jj, carry2):
                k0 = 2 * jj
                pltpu.async_copy(b_src(b0, k0 + 1), xb1, sb1)
                pltpu.make_async_copy(b_src(b0, k0), xb0, sb0).wait()

                @pl.when(jj > 0)
                def _():
                    pltpu.make_async_copy(nb0, lat_dst(b0, k0 - 2),
                                          so0).wait()

                fill_nb(nb0, xb0)
                pltpu.async_copy(nb0, lat_dst(b0, k0), so0)

                @pl.when(jj < nk // 2 - 1)
                def _():
                    pltpu.async_copy(b_src(b0, k0 + 2), xb0, sb0)

                pltpu.make_async_copy(b_src(b0, k0 + 1), xb1, sb1).wait()

                @pl.when(jj > 0)
                def _():
                    pltpu.make_async_copy(nb1, lat_dst(b0, k0 - 1),
                                          so1).wait()

                fill_nb(nb1, xb1)
                pltpu.async_copy(nb1, lat_dst(b0, k0 + 1), so1)
                return carry2

            lax.fori_loop(0, nk // 2, bj, 0)

            pltpu.make_async_copy(nb0, lat_dst(b0, nk - 2), so0).wait()
            pltpu.make_async_copy(nb1, lat_dst(b0, nk - 1), so1).wait()
            for g in range(_NG):
                giv = gi_buf[g, :]
                rows = lanes + g * 16
                for cl in range(_CS):
                    plsc.store_scatter(nb0, [rows, giv + cl * D], zero16)
                    plsc.store_scatter(nb1, [rows, giv + cl * D], zero16)
            return carry

        lax.fori_loop(0, b_per_w // _NB, chunk, 0)

    logt, lat = run(xt)
    return (jnp.transpose(logt), lat)
